# baseline (speedup 1.0000x reference)
import functools

import jax
import jax.numpy as jnp
from jax import lax
from jax.experimental import pallas as pl
from jax.experimental.pallas import tpu as pltpu
from jax.experimental.pallas import tpu_sc as plsc

N = 10000
NP = 10240
E = 160000
IN = 128
H = 16
EMB = 64
B = 64
ITERS = 3

NC = 2
NS = 16
NW = NC * NS
CH = 128
NCH = 40
EPW = CH * NCH
EP = NW * EPW

ET = 2048
NT = EP // ET
NBLK = 10
BLK = NP // NBLK
RPS = NP // NS

f32 = jnp.float32


def _gather_body(s_hbm, src_hbm, g_hbm, idx_v, rows_v, tab):
    c = lax.axis_index("c")
    s = lax.axis_index("s")
    wid = s * NC + c
    pltpu.sync_copy(s_hbm.at[pl.ds(s * RPS, RPS)], tab.at[pl.ds(s * RPS, RPS)])
    pltpu.sync_copy(src_hbm.at[wid], idx_v)
    plsc.subcore_barrier()

    def chunk(jo, carry):
        for ji in range(8):
            j = jo * 8 + ji
            pltpu.sync_copy(tab.at[idx_v.at[j]], rows_v.at[pl.ds(j * CH, CH)])
        return carry

    lax.fori_loop(0, NCH // 8, chunk, 0)
    pltpu.sync_copy(rows_v, g_hbm.at[pl.ds(wid * EPW, EPW)])


@functools.cache
def _sc_gather_kernel():
    mesh = plsc.VectorSubcoreMesh(core_axis_name="c", subcore_axis_name="s",
                                  num_cores=NC, num_subcores=NS)
    return functools.partial(
        pl.kernel,
        mesh=mesh,
        compiler_params=pltpu.CompilerParams(use_tc_tiling_on_sc=False),
        out_type=jax.ShapeDtypeStruct((EP, H), f32),
        scratch_types=[
            pltpu.VMEM((NCH, CH), jnp.int32),
            pltpu.VMEM((EPW, H), f32),
            pltpu.VMEM_SHARED((NP, H), f32),
        ],
    )(_gather_body)


def _sc_gather(s, src3):
    return _sc_gather_kernel()(s, src3)


def _scatter_body(msg_hbm, dst_hbm, zero_hbm, agg_hbm, idx_v, rows_v, acc):
    c = lax.axis_index("c")
    s = lax.axis_index("s")
    wid = s * NC + c
    pltpu.sync_copy(zero_hbm.at[pl.ds(s * RPS, RPS)], acc.at[pl.ds(s * RPS, RPS)])
    pltpu.sync_copy(dst_hbm.at[wid], idx_v)
    pltpu.sync_copy(msg_hbm.at[pl.ds(wid * EPW, EPW)], rows_v)
    plsc.subcore_barrier()

    def chunk(jo, carry):
        for ji in range(8):
            j = jo * 8 + ji
            pltpu.sync_copy(rows_v.at[pl.ds(j * CH, CH)], acc.at[idx_v.at[j]],
                            add=True)
        return carry

    lax.fori_loop(0, NCH // 8, chunk, 0)
    plsc.subcore_barrier()
    pltpu.sync_copy(acc.at[pl.ds(s * RPS, RPS)], agg_hbm.at[c, pl.ds(s * RPS, RPS)])


@functools.cache
def _sc_scatter_kernel():
    mesh = plsc.VectorSubcoreMesh(core_axis_name="c", subcore_axis_name="s",
                                  num_cores=NC, num_subcores=NS)
    return functools.partial(
        pl.kernel,
        mesh=mesh,
        compiler_params=pltpu.CompilerParams(use_tc_tiling_on_sc=False),
        out_type=jax.ShapeDtypeStruct((NC, NP, H), f32),
        scratch_types=[
            pltpu.VMEM((NCH, CH), jnp.int32),
            pltpu.VMEM((EPW, H), f32),
            pltpu.VMEM_SHARED((NP, H), f32),
        ],
    )(_scatter_body)


def _sc_scatter(msg, dst3, zero_n):
    return _sc_scatter_kernel()(msg, dst3, zero_n)


def _in_fc_body(x_ref, w_ref, b_ref, o_ref):
    o_ref[...] = jnp.maximum(
        jnp.dot(x_ref[...], w_ref[...], preferred_element_type=f32)
        + b_ref[...], 0.0)


def _in_fc(x, w_t, b):
    return pl.pallas_call(
        _in_fc_body,
        grid=(NBLK,),
        in_specs=[
            pl.BlockSpec((BLK, IN), lambda i: (i, 0)),
            pl.BlockSpec((IN, H), lambda i: (0, 0)),
            pl.BlockSpec((1, H), lambda i: (0, 0)),
        ],
        out_specs=pl.BlockSpec((BLK, H), lambda i: (i, 0)),
        out_shape=jax.ShapeDtypeStruct((NP, H), f32),
    )(x, w_t, b)


def _msg_body(ea_ref, g_ref, w1_ref, b1_ref, w2_ref, b2_ref, r_ref, s_ref,
              msg_ref):
    t = pl.program_id(0)
    bf = jnp.bfloat16
    a = jnp.maximum(
        jnp.dot(ea_ref[...].astype(bf), w1_ref[...].astype(bf),
                preferred_element_type=f32) + b1_ref[...], 0.0)
    we = jnp.dot(a.astype(bf), w2_ref[...].astype(bf),
                 preferred_element_type=f32) + b2_ref[...]
    grep = jnp.dot(g_ref[...].astype(bf), r_ref[...].astype(bf),
                   preferred_element_type=f32)
    msg = jnp.dot((grep * we).astype(bf), s_ref[...].astype(bf),
                  preferred_element_type=f32)
    eid = t * ET + lax.broadcasted_iota(jnp.int32, (ET, 1), 0)
    msg_ref[...] = jnp.where(eid < E, msg, 0.0)


def _msg(ea_p, g, w1_t, b1, w2_t, b2, r_m, s_m):
    return pl.pallas_call(
        _msg_body,
        grid=(NT,),
        in_specs=[
            pl.BlockSpec((ET, 4), lambda i: (i, 0)),
            pl.BlockSpec((ET, H), lambda i: (i, 0)),
            pl.BlockSpec((4, 128), lambda i: (0, 0)),
            pl.BlockSpec((1, 128), lambda i: (0, 0)),
            pl.BlockSpec((128, H * H), lambda i: (0, 0)),
            pl.BlockSpec((1, H * H), lambda i: (0, 0)),
            pl.BlockSpec((H, H * H), lambda i: (0, 0)),
            pl.BlockSpec((H * H, H), lambda i: (0, 0)),
        ],
        out_specs=pl.BlockSpec((ET, H), lambda i: (i, 0)),
        out_shape=jax.ShapeDtypeStruct((EP, H), f32),
    )(ea_p, g, w1_t, b1, w2_t, b2, r_m, s_m)


def _node_body(s_ref, a0_ref, a1_ref, wroot_ref, bconv_ref,
               wir_ref, wiz_ref, win_ref, bi_ref,
               whr_ref, whz_ref, whn_ref, bh_ref, o_ref):
    sv = s_ref[...]
    agg = a0_ref[...] + a1_ref[...]
    m = jnp.maximum(
        jnp.dot(sv, wroot_ref[...], preferred_element_type=f32) + agg
        + bconv_ref[...], 0.0)
    bi = bi_ref[...]
    bh = bh_ref[...]
    gir = jnp.dot(m, wir_ref[...], preferred_element_type=f32) + bi[:, :H]
    giz = jnp.dot(m, wiz_ref[...], preferred_element_type=f32) + bi[:, H:2 * H]
    gin = jnp.dot(m, win_ref[...], preferred_element_type=f32) + bi[:, 2 * H:]
    ghr = jnp.dot(sv, whr_ref[...], preferred_element_type=f32) + bh[:, :H]
    ghz = jnp.dot(sv, whz_ref[...], preferred_element_type=f32) + bh[:, H:2 * H]
    ghn = jnp.dot(sv, whn_ref[...], preferred_element_type=f32) + bh[:, 2 * H:]
    r = jax.nn.sigmoid(gir + ghr)
    z = jax.nn.sigmoid(giz + ghz)
    n = jnp.tanh(gin + r * ghn)
    o_ref[...] = (1.0 - z) * n + z * sv


def _node(s, agg0, agg1, wroot_t, bconv, wir, wiz, win, bi, whr, whz, whn, bh):
    wspec = pl.BlockSpec((H, H), lambda i: (0, 0))
    bspec = pl.BlockSpec((1, 3 * H), lambda i: (0, 0))
    nspec = pl.BlockSpec((BLK, H), lambda i: (i, 0))
    return pl.pallas_call(
        _node_body,
        grid=(NBLK,),
        in_specs=[
            nspec, nspec, nspec,
            wspec, pl.BlockSpec((1, H), lambda i: (0, 0)),
            wspec, wspec, wspec, bspec,
            wspec, wspec, wspec, bspec,
        ],
        out_specs=nspec,
        out_shape=jax.ShapeDtypeStruct((NP, H), f32),
    )(s, agg0, agg1, wroot_t, bconv, wir, wiz, win, bi, whr, whz, whn, bh)


def _epi_body(s_ref, x_ref, bat_ref, w1_ref, b1_ref, w2_ref, b2_ref,
              fg1_ref, fg2_ref, seg1_ref, seg2_ref, cnt_ref):
    i = pl.program_id(0)
    hmid = jnp.maximum(
        jnp.dot(s_ref[...], w1_ref[...], preferred_element_type=f32)
        + b1_ref[...], 0.0)
    emb = jnp.dot(hmid, w2_ref[...], preferred_element_type=f32) + b2_ref[...]
    xv = x_ref[...]
    ss = (jnp.sum(emb * emb, axis=1, keepdims=True)
          + jnp.sum(xv * xv, axis=1, keepdims=True))
    inv = 1.0 / jnp.maximum(jnp.sqrt(ss), 1e-12)
    fg1 = emb * inv
    fg2 = xv * inv
    fg1_ref[...] = fg1
    fg2_ref[...] = fg2
    onehot = (bat_ref[...] == lax.broadcasted_iota(jnp.int32, (BLK, B), 1)
              ).astype(f32)
    dn = (((0,), (0,)), ((), ()))
    p1 = lax.dot_general(onehot, fg1, dn, preferred_element_type=f32)
    p2 = lax.dot_general(onehot, fg2, dn, preferred_element_type=f32)
    pc = jnp.broadcast_to(jnp.sum(onehot, axis=0)[:, None], (B, IN))

    @pl.when(i == 0)
    def _():
        seg1_ref[...] = jnp.zeros_like(seg1_ref)
        seg2_ref[...] = jnp.zeros_like(seg2_ref)
        cnt_ref[...] = jnp.zeros_like(cnt_ref)

    seg1_ref[...] += p1
    seg2_ref[...] += p2
    cnt_ref[...] += pc


def _epilogue(s, x, bat2, w1_t, b1, w2_t, b2):
    zspec = lambda shape: pl.BlockSpec(shape, lambda i: (0, 0))
    return pl.pallas_call(
        _epi_body,
        grid=(NBLK,),
        in_specs=[
            pl.BlockSpec((BLK, H), lambda i: (i, 0)),
            pl.BlockSpec((BLK, IN), lambda i: (i, 0)),
            pl.BlockSpec((BLK, 1), lambda i: (i, 0)),
            zspec((H, H)), zspec((1, H)), zspec((H, EMB)), zspec((1, EMB)),
        ],
        out_specs=[
            pl.BlockSpec((BLK, EMB), lambda i: (i, 0)),
            pl.BlockSpec((BLK, IN), lambda i: (i, 0)),
            zspec((B, EMB)), zspec((B, IN)), zspec((B, IN)),
        ],
        out_shape=[
            jax.ShapeDtypeStruct((NP, EMB), f32),
            jax.ShapeDtypeStruct((NP, IN), f32),
            jax.ShapeDtypeStruct((B, EMB), f32),
            jax.ShapeDtypeStruct((B, IN), f32),
            jax.ShapeDtypeStruct((B, IN), f32),
        ],
    )(s, x, bat2, w1_t, b1, w2_t, b2)


def _ratio_body(s1_ref, s2_ref, cnt_ref, wp1_ref, wp2_ref, bp_ref, o_ref):
    cnt = jnp.maximum(cnt_ref[...], 1.0)
    r1 = s1_ref[...] / cnt[:, :EMB]
    r2 = s2_ref[...] / cnt
    v = (jnp.dot(r1, wp1_ref[...], preferred_element_type=f32)
         + jnp.dot(r2, wp2_ref[...], preferred_element_type=f32)
         + bp_ref[...])
    o_ref[...] = jax.nn.sigmoid(v)


def _ratio(seg1, seg2, cnt, wp1_t, wp2_t, bp):
    return pl.pallas_call(
        _ratio_body,
        out_shape=jax.ShapeDtypeStruct((B, 1), f32),
    )(seg1, seg2, cnt, wp1_t, wp2_t, bp)


def kernel(x, edge_attr, W_in_fc, b_in_fc, W_e1, b_e1, W_e2, b_e2,
           W_root, b_conv, W_ih, W_hh, b_ih, b_hh,
           W_o1, b_o1, W_o2, b_o2, W_p, b_p,
           edge_index, batch):
    src = jnp.zeros((EP,), jnp.int32).at[:E].set(edge_index[0])
    dst = jnp.zeros((EP,), jnp.int32).at[:E].set(edge_index[1])
    src3 = src.reshape(NW, NCH, CH)
    dst3 = dst.reshape(NW, NCH, CH)
    ea_p = jnp.zeros((EP, 4), f32).at[:E].set(edge_attr)
    zero_n = jnp.zeros((NP, H), f32)
    x_p = jnp.zeros((NP, IN), f32).at[:N].set(x)
    bat2 = jnp.full((NP, 1), B, jnp.int32).at[:N, 0].set(batch)

    r_m = jnp.repeat(jnp.eye(H, dtype=f32), H, axis=1)
    s_m = jnp.tile(jnp.eye(H, dtype=f32), (H, 1))

    w_in_t = W_in_fc.T
    w1_t = W_e1.T
    w2_t = W_e2.T
    wroot_t = W_root.T
    wir, wiz, win = (W_ih[:H].T, W_ih[H:2 * H].T, W_ih[2 * H:].T)
    whr, whz, whn = (W_hh[:H].T, W_hh[H:2 * H].T, W_hh[2 * H:].T)
    wo1_t = W_o1.T
    wo2_t = W_o2.T
    wp1_t = W_p[:, :EMB].T
    wp2_t = W_p[:, EMB:].T

    b_in = b_in_fc.reshape(1, H)
    b1 = b_e1.reshape(1, 128)
    b2 = b_e2.reshape(1, H * H)
    bconv = b_conv.reshape(1, H)
    bi = b_ih.reshape(1, 3 * H)
    bh = b_hh.reshape(1, 3 * H)
    bo1 = b_o1.reshape(1, H)
    bo2 = b_o2.reshape(1, EMB)
    bp = b_p.reshape(1, 1)

    s = _in_fc(x_p, w_in_t, b_in)
    for _ in range(ITERS):
        g = _sc_gather(s, src3)
        msg = _msg(ea_p, g, w1_t, b1, w2_t, b2, r_m, s_m)
        agg2 = _sc_scatter(msg, dst3, zero_n)
        s = _node(s, agg2[0], agg2[1], wroot_t, bconv, wir, wiz, win, bi,
                  whr, whz, whn, bh)
    fg1, fg2, seg1, seg2, cnt = _epilogue(s, x_p, bat2, wo1_t, bo1, wo2_t, bo2)
    fg_embed = jnp.concatenate([fg1[:N], fg2[:N]], axis=1)
    cg_fg_ratio = _ratio(seg1, seg2, cnt, wp1_t, wp2_t, bp)
    return (fg_embed, cg_fg_ratio)


# speedup vs baseline: 1.5828x; 1.5828x over previous
import functools

import jax
import jax.numpy as jnp
from jax import lax
from jax.experimental import pallas as pl
from jax.experimental.pallas import tpu as pltpu
from jax.experimental.pallas import tpu_sc as plsc

N = 10000
NP = 10240
E = 160000
IN = 128
H = 16
EMB = 64
B = 64
ITERS = 3

NC = 2
NS = 16
NW = NC * NS
CH = 128
NCH = 40
EPW = CH * NCH
EP = NW * EPW

MB = 512
NBLK = 10
BLK = NP // NBLK
RPS = NP // NS

f32 = jnp.float32


def _gather_body(s_hbm, src_hbm, g_hbm, idx_v, rows_v, tab):
    c = lax.axis_index("c")
    s = lax.axis_index("s")
    wid = s * NC + c
    pltpu.sync_copy(s_hbm.at[pl.ds(s * RPS, RPS)], tab.at[pl.ds(s * RPS, RPS)])
    pltpu.sync_copy(src_hbm.at[wid], idx_v)
    plsc.subcore_barrier()

    def chunk(jo, carry):
        for ji in range(8):
            j = jo * 8 + ji
            pltpu.sync_copy(tab.at[idx_v.at[j]], rows_v.at[pl.ds(j * CH, CH)])
        return carry

    lax.fori_loop(0, NCH // 8, chunk, 0)
    pltpu.sync_copy(rows_v, g_hbm.at[pl.ds(wid * EPW, EPW)])


@functools.cache
def _sc_gather_kernel():
    mesh = plsc.VectorSubcoreMesh(core_axis_name="c", subcore_axis_name="s",
                                  num_cores=NC, num_subcores=NS)
    return functools.partial(
        pl.kernel,
        mesh=mesh,
        compiler_params=pltpu.CompilerParams(use_tc_tiling_on_sc=False),
        out_type=jax.ShapeDtypeStruct((EP, H), f32),
        scratch_types=[
            pltpu.VMEM((NCH, CH), jnp.int32),
            pltpu.VMEM((EPW, H), f32),
            pltpu.VMEM_SHARED((NP, H), f32),
        ],
    )(_gather_body)


def _sc_gather(s, src3):
    return _sc_gather_kernel()(s, src3)


def _scatter_body(msg_hbm, dst_hbm, zero_hbm, agg_hbm, idx_v, rows_v, acc):
    c = lax.axis_index("c")
    s = lax.axis_index("s")
    wid = s * NC + c
    pltpu.sync_copy(zero_hbm.at[pl.ds(s * RPS, RPS)], acc.at[pl.ds(s * RPS, RPS)])
    pltpu.sync_copy(dst_hbm.at[wid], idx_v)
    pltpu.sync_copy(msg_hbm.at[pl.ds(wid * EPW, EPW)], rows_v)
    plsc.subcore_barrier()

    def chunk(jo, carry):
        for ji in range(8):
            j = jo * 8 + ji
            pltpu.sync_copy(rows_v.at[pl.ds(j * CH, CH)], acc.at[idx_v.at[j]],
                            add=True)
        return carry

    lax.fori_loop(0, NCH // 8, chunk, 0)
    plsc.subcore_barrier()
    pltpu.sync_copy(acc.at[pl.ds(s * RPS, RPS)], agg_hbm.at[c, pl.ds(s * RPS, RPS)])


@functools.cache
def _sc_scatter_kernel():
    mesh = plsc.VectorSubcoreMesh(core_axis_name="c", subcore_axis_name="s",
                                  num_cores=NC, num_subcores=NS)
    return functools.partial(
        pl.kernel,
        mesh=mesh,
        compiler_params=pltpu.CompilerParams(use_tc_tiling_on_sc=False),
        out_type=jax.ShapeDtypeStruct((NC, NP, H), f32),
        scratch_types=[
            pltpu.VMEM((NCH, CH), jnp.int32),
            pltpu.VMEM((EPW, H), f32),
            pltpu.VMEM_SHARED((NP, H), f32),
        ],
    )(_scatter_body)


def _sc_scatter(msg, dst3, zero_n):
    return _sc_scatter_kernel()(msg, dst3, zero_n)


def _in_fc_body(x_ref, w_ref, b_ref, o_ref):
    o_ref[...] = jnp.maximum(
        jnp.dot(x_ref[...], w_ref[...], preferred_element_type=f32)
        + b_ref[...], 0.0)


def _in_fc(x, w_t, b):
    return pl.pallas_call(
        _in_fc_body,
        grid=(NBLK,),
        in_specs=[
            pl.BlockSpec((BLK, IN), lambda i: (i, 0)),
            pl.BlockSpec((IN, H), lambda i: (0, 0)),
            pl.BlockSpec((1, H), lambda i: (0, 0)),
        ],
        out_specs=pl.BlockSpec((BLK, H), lambda i: (i, 0)),
        out_shape=jax.ShapeDtypeStruct((NP, H), f32),
    )(x, w_t, b)


def _msg_body(ea_ref, g_ref, w1_ref, b1_ref, w2_ref, b2_ref, r_ref, s_ref,
              msg_ref):
    t = pl.program_id(0)
    bf = jnp.bfloat16
    rmask = (t * MB + lax.broadcasted_iota(jnp.int32, (MB, 1), 0)) < E // 8
    w1 = w1_ref[...].astype(bf)
    w2 = w2_ref[...].astype(bf)
    r_m = r_ref[...].astype(bf)
    s_m = s_ref[...].astype(bf)
    b1 = b1_ref[...]
    b2 = b2_ref[...]
    for p in range(8):
        ea_p = ea_ref[:, 4 * p:4 * p + 4].astype(bf)
        g_p = g_ref[:, H * p:H * p + H].astype(bf)
        a = jnp.maximum(
            jnp.dot(ea_p, w1, preferred_element_type=f32) + b1, 0.0)
        we = jnp.dot(a.astype(bf), w2, preferred_element_type=f32) + b2
        grep = jnp.dot(g_p, r_m, preferred_element_type=f32)
        msg = jnp.dot((grep * we).astype(bf), s_m, preferred_element_type=f32)
        msg_ref[:, H * p:H * p + H] = jnp.where(rmask, msg, 0.0)


def _msg(ea32, g128, w1_t, b1, w2_t, b2, r_m, s_m):
    return pl.pallas_call(
        _msg_body,
        grid=(EP // 8 // MB,),
        in_specs=[
            pl.BlockSpec((MB, 32), lambda i: (i, 0)),
            pl.BlockSpec((MB, 128), lambda i: (i, 0)),
            pl.BlockSpec((4, 128), lambda i: (0, 0)),
            pl.BlockSpec((1, 128), lambda i: (0, 0)),
            pl.BlockSpec((128, H * H), lambda i: (0, 0)),
            pl.BlockSpec((1, H * H), lambda i: (0, 0)),
            pl.BlockSpec((H, H * H), lambda i: (0, 0)),
            pl.BlockSpec((H * H, H), lambda i: (0, 0)),
        ],
        out_specs=pl.BlockSpec((MB, 128), lambda i: (i, 0)),
        out_shape=jax.ShapeDtypeStruct((EP // 8, 128), f32),
    )(ea32, g128, w1_t, b1, w2_t, b2, r_m, s_m)


def _node_body(s_ref, a0_ref, a1_ref, wroot_ref, bconv_ref,
               wir_ref, wiz_ref, win_ref, bi_ref,
               whr_ref, whz_ref, whn_ref, bh_ref, o_ref):
    sv = s_ref[...]
    agg = a0_ref[...] + a1_ref[...]
    m = jnp.maximum(
        jnp.dot(sv, wroot_ref[...], preferred_element_type=f32) + agg
        + bconv_ref[...], 0.0)
    bi = bi_ref[...]
    bh = bh_ref[...]
    gir = jnp.dot(m, wir_ref[...], preferred_element_type=f32) + bi[:, :H]
    giz = jnp.dot(m, wiz_ref[...], preferred_element_type=f32) + bi[:, H:2 * H]
    gin = jnp.dot(m, win_ref[...], preferred_element_type=f32) + bi[:, 2 * H:]
    ghr = jnp.dot(sv, whr_ref[...], preferred_element_type=f32) + bh[:, :H]
    ghz = jnp.dot(sv, whz_ref[...], preferred_element_type=f32) + bh[:, H:2 * H]
    ghn = jnp.dot(sv, whn_ref[...], preferred_element_type=f32) + bh[:, 2 * H:]
    r = jax.nn.sigmoid(gir + ghr)
    z = jax.nn.sigmoid(giz + ghz)
    n = jnp.tanh(gin + r * ghn)
    o_ref[...] = (1.0 - z) * n + z * sv


def _node(s, agg0, agg1, wroot_t, bconv, wir, wiz, win, bi, whr, whz, whn, bh):
    wspec = pl.BlockSpec((H, H), lambda i: (0, 0))
    bspec = pl.BlockSpec((1, 3 * H), lambda i: (0, 0))
    nspec = pl.BlockSpec((BLK, H), lambda i: (i, 0))
    return pl.pallas_call(
        _node_body,
        grid=(NBLK,),
        in_specs=[
            nspec, nspec, nspec,
            wspec, pl.BlockSpec((1, H), lambda i: (0, 0)),
            wspec, wspec, wspec, bspec,
            wspec, wspec, wspec, bspec,
        ],
        out_specs=nspec,
        out_shape=jax.ShapeDtypeStruct((NP, H), f32),
    )(s, agg0, agg1, wroot_t, bconv, wir, wiz, win, bi, whr, whz, whn, bh)


def _epi_body(s_ref, x_ref, bat_ref, w1_ref, b1_ref, w2_ref, b2_ref,
              fg1_ref, fg2_ref, seg1_ref, seg2_ref, cnt_ref):
    i = pl.program_id(0)
    hmid = jnp.maximum(
        jnp.dot(s_ref[...], w1_ref[...], preferred_element_type=f32)
        + b1_ref[...], 0.0)
    emb = jnp.dot(hmid, w2_ref[...], preferred_element_type=f32) + b2_ref[...]
    xv = x_ref[...]
    ss = (jnp.sum(emb * emb, axis=1, keepdims=True)
          + jnp.sum(xv * xv, axis=1, keepdims=True))
    inv = 1.0 / jnp.maximum(jnp.sqrt(ss), 1e-12)
    fg1 = emb * inv
    fg2 = xv * inv
    fg1_ref[...] = fg1
    fg2_ref[...] = fg2
    onehot = (bat_ref[...] == lax.broadcasted_iota(jnp.int32, (BLK, B), 1)
              ).astype(f32)
    dn = (((0,), (0,)), ((), ()))
    p1 = lax.dot_general(onehot, fg1, dn, preferred_element_type=f32)
    p2 = lax.dot_general(onehot, fg2, dn, preferred_element_type=f32)
    pc = jnp.broadcast_to(jnp.sum(onehot, axis=0)[:, None], (B, IN))

    @pl.when(i == 0)
    def _():
        seg1_ref[...] = jnp.zeros_like(seg1_ref)
        seg2_ref[...] = jnp.zeros_like(seg2_ref)
        cnt_ref[...] = jnp.zeros_like(cnt_ref)

    seg1_ref[...] += p1
    seg2_ref[...] += p2
    cnt_ref[...] += pc


def _epilogue(s, x, bat2, w1_t, b1, w2_t, b2):
    zspec = lambda shape: pl.BlockSpec(shape, lambda i: (0, 0))
    return pl.pallas_call(
        _epi_body,
        grid=(NBLK,),
        in_specs=[
            pl.BlockSpec((BLK, H), lambda i: (i, 0)),
            pl.BlockSpec((BLK, IN), lambda i: (i, 0)),
            pl.BlockSpec((BLK, 1), lambda i: (i, 0)),
            zspec((H, H)), zspec((1, H)), zspec((H, EMB)), zspec((1, EMB)),
        ],
        out_specs=[
            pl.BlockSpec((BLK, EMB), lambda i: (i, 0)),
            pl.BlockSpec((BLK, IN), lambda i: (i, 0)),
            zspec((B, EMB)), zspec((B, IN)), zspec((B, IN)),
        ],
        out_shape=[
            jax.ShapeDtypeStruct((NP, EMB), f32),
            jax.ShapeDtypeStruct((NP, IN), f32),
            jax.ShapeDtypeStruct((B, EMB), f32),
            jax.ShapeDtypeStruct((B, IN), f32),
            jax.ShapeDtypeStruct((B, IN), f32),
        ],
    )(s, x, bat2, w1_t, b1, w2_t, b2)


def _ratio_body(s1_ref, s2_ref, cnt_ref, wp1_ref, wp2_ref, bp_ref, o_ref):
    cnt = jnp.maximum(cnt_ref[...], 1.0)
    r1 = s1_ref[...] / cnt[:, :EMB]
    r2 = s2_ref[...] / cnt
    v = (jnp.dot(r1, wp1_ref[...], preferred_element_type=f32)
         + jnp.dot(r2, wp2_ref[...], preferred_element_type=f32)
         + bp_ref[...])
    o_ref[...] = jax.nn.sigmoid(v)


def _ratio(seg1, seg2, cnt, wp1_t, wp2_t, bp):
    return pl.pallas_call(
        _ratio_body,
        out_shape=jax.ShapeDtypeStruct((B, 1), f32),
    )(seg1, seg2, cnt, wp1_t, wp2_t, bp)


def kernel(x, edge_attr, W_in_fc, b_in_fc, W_e1, b_e1, W_e2, b_e2,
           W_root, b_conv, W_ih, W_hh, b_ih, b_hh,
           W_o1, b_o1, W_o2, b_o2, W_p, b_p,
           edge_index, batch):
    src = jnp.zeros((EP,), jnp.int32).at[:E].set(edge_index[0])
    dst = jnp.zeros((EP,), jnp.int32).at[:E].set(edge_index[1])
    src3 = src.reshape(NW, NCH, CH)
    dst3 = dst.reshape(NW, NCH, CH)
    ea32 = jnp.zeros((EP // 8, 32), f32).at[:E // 8].set(
        edge_attr.reshape(E // 8, 32))
    zero_n = jnp.zeros((NP, H), f32)
    x_p = jnp.zeros((NP, IN), f32).at[:N].set(x)
    bat2 = jnp.full((NP, 1), B, jnp.int32).at[:N, 0].set(batch)

    r_m = jnp.repeat(jnp.eye(H, dtype=f32), H, axis=1)
    s_m = jnp.tile(jnp.eye(H, dtype=f32), (H, 1))

    w_in_t = W_in_fc.T
    w1_t = W_e1.T
    w2_t = W_e2.T
    wroot_t = W_root.T
    wir, wiz, win = (W_ih[:H].T, W_ih[H:2 * H].T, W_ih[2 * H:].T)
    whr, whz, whn = (W_hh[:H].T, W_hh[H:2 * H].T, W_hh[2 * H:].T)
    wo1_t = W_o1.T
    wo2_t = W_o2.T
    wp1_t = W_p[:, :EMB].T
    wp2_t = W_p[:, EMB:].T

    b_in = b_in_fc.reshape(1, H)
    b1 = b_e1.reshape(1, 128)
    b2 = b_e2.reshape(1, H * H)
    bconv = b_conv.reshape(1, H)
    bi = b_ih.reshape(1, 3 * H)
    bh = b_hh.reshape(1, 3 * H)
    bo1 = b_o1.reshape(1, H)
    bo2 = b_o2.reshape(1, EMB)
    bp = b_p.reshape(1, 1)

    s = _in_fc(x_p, w_in_t, b_in)
    for _ in range(ITERS):
        g = _sc_gather(s, src3)
        msg128 = _msg(ea32, g.reshape(EP // 8, 128), w1_t, b1, w2_t, b2,
                      r_m, s_m)
        agg2 = _sc_scatter(msg128.reshape(EP, H), dst3, zero_n)
        s = _node(s, agg2[0], agg2[1], wroot_t, bconv, wir, wiz, win, bi,
                  whr, whz, whn, bh)
    fg1, fg2, seg1, seg2, cnt = _epilogue(s, x_p, bat2, wo1_t, bo1, wo2_t, bo2)
    fg_embed = jnp.concatenate([fg1[:N], fg2[:N]], axis=1)
    cg_fg_ratio = _ratio(seg1, seg2, cnt, wp1_t, wp2_t, bp)
    return (fg_embed, cg_fg_ratio)


# speedup vs baseline: 1.8146x; 1.1464x over previous
import functools

import jax
import jax.numpy as jnp
from jax import lax
from jax.experimental import pallas as pl
from jax.experimental.pallas import tpu as pltpu
from jax.experimental.pallas import tpu_sc as plsc

N = 10000
NP = 10240
E = 160000
IN = 128
H = 16
EMB = 64
B = 64
ITERS = 3

NC = 2
NS = 16
NW = NC * NS
CH = 128
NCH = 40
EPW = CH * NCH
EP = NW * EPW

MB = 512
NBLK = 10
BLK = NP // NBLK
RPS = NP // NS

f32 = jnp.float32


def _gather_body(s_hbm, src_hbm, g_hbm, idx_v, rows_v, tab):
    c = lax.axis_index("c")
    s = lax.axis_index("s")
    wid = s * NC + c
    pltpu.sync_copy(s_hbm.at[pl.ds(s * RPS, RPS)], tab.at[pl.ds(s * RPS, RPS)])
    pltpu.sync_copy(src_hbm.at[wid], idx_v)
    plsc.subcore_barrier()

    def chunk(jo, carry):
        for ji in range(8):
            j = jo * 8 + ji
            pltpu.sync_copy(tab.at[idx_v.at[j]], rows_v.at[pl.ds(j * CH, CH)])
        return carry

    lax.fori_loop(0, NCH // 8, chunk, 0)
    pltpu.sync_copy(rows_v, g_hbm.at[pl.ds(wid * EPW, EPW)])


@functools.cache
def _sc_gather_kernel():
    mesh = plsc.VectorSubcoreMesh(core_axis_name="c", subcore_axis_name="s",
                                  num_cores=NC, num_subcores=NS)
    return functools.partial(
        pl.kernel,
        mesh=mesh,
        compiler_params=pltpu.CompilerParams(use_tc_tiling_on_sc=False),
        out_type=jax.ShapeDtypeStruct((EP, H), f32),
        scratch_types=[
            pltpu.VMEM((NCH, CH), jnp.int32),
            pltpu.VMEM((EPW, H), f32),
            pltpu.VMEM_SHARED((NP, H), f32),
        ],
    )(_gather_body)


def _sc_gather(s, src3):
    return _sc_gather_kernel()(s, src3)


def _scatter_body(msg_hbm, dst_hbm, zero_hbm, agg_hbm, idx_v, rows_v, acc):
    c = lax.axis_index("c")
    s = lax.axis_index("s")
    wid = s * NC + c
    pltpu.sync_copy(zero_hbm.at[pl.ds(s * RPS, RPS)], acc.at[pl.ds(s * RPS, RPS)])
    pltpu.sync_copy(dst_hbm.at[wid], idx_v)
    pltpu.sync_copy(msg_hbm.at[pl.ds(wid * EPW, EPW)], rows_v)
    plsc.subcore_barrier()

    def chunk(jo, carry):
        for ji in range(8):
            j = jo * 8 + ji
            pltpu.sync_copy(rows_v.at[pl.ds(j * CH, CH)], acc.at[idx_v.at[j]],
                            add=True)
        return carry

    lax.fori_loop(0, NCH // 8, chunk, 0)
    plsc.subcore_barrier()
    pltpu.sync_copy(acc.at[pl.ds(s * RPS, RPS)], agg_hbm.at[c, pl.ds(s * RPS, RPS)])


@functools.cache
def _sc_scatter_kernel():
    mesh = plsc.VectorSubcoreMesh(core_axis_name="c", subcore_axis_name="s",
                                  num_cores=NC, num_subcores=NS)
    return functools.partial(
        pl.kernel,
        mesh=mesh,
        compiler_params=pltpu.CompilerParams(use_tc_tiling_on_sc=False),
        out_type=jax.ShapeDtypeStruct((NC, NP, H), f32),
        scratch_types=[
            pltpu.VMEM((NCH, CH), jnp.int32),
            pltpu.VMEM((EPW, H), f32),
            pltpu.VMEM_SHARED((NP, H), f32),
        ],
    )(_scatter_body)


def _sc_scatter(msg, dst3, zero_n):
    return _sc_scatter_kernel()(msg, dst3, zero_n)


def _in_fc_body(x_ref, w_ref, b_ref, o_ref):
    o_ref[...] = jnp.maximum(
        jnp.dot(x_ref[...], w_ref[...], preferred_element_type=f32)
        + b_ref[...], 0.0)


def _in_fc(x, w_t, b):
    return pl.pallas_call(
        _in_fc_body,
        grid=(NBLK,),
        in_specs=[
            pl.BlockSpec((BLK, IN), lambda i: (i, 0)),
            pl.BlockSpec((IN, H), lambda i: (0, 0)),
            pl.BlockSpec((1, H), lambda i: (0, 0)),
        ],
        out_specs=pl.BlockSpec((BLK, H), lambda i: (i, 0)),
        out_shape=jax.ShapeDtypeStruct((NP, H), f32),
    )(x, w_t, b)


def _msg_body(ea_ref, g_ref, w1_ref, b1_ref, w2_ref, b2_ref, s_ref,
              msg_ref):
    t = pl.program_id(0)
    bf = jnp.bfloat16
    rmask = (t * MB + lax.broadcasted_iota(jnp.int32, (MB, 1), 0)) < E // 8
    abd = jnp.maximum(
        jnp.dot(ea_ref[...].astype(bf), w1_ref[...].astype(bf),
                preferred_element_type=f32) + b1_ref[...], 0.0).astype(bf)
    w2 = w2_ref[...].astype(bf)
    s_m = s_ref[...].astype(bf)
    gbf = g_ref[...].astype(bf)
    for p in range(8):
        we = (jnp.dot(abd[:, 128 * p:128 * p + 128], w2,
                      preferred_element_type=f32) + b2_ref[...]).astype(bf)
        grep = jnp.tile(gbf[:, H * p:H * p + H], (1, H))
        msg = jnp.dot(grep * we, s_m, preferred_element_type=f32)
        msg_ref[:, H * p:H * p + H] = jnp.where(rmask, msg, 0.0)


def _msg(ea32, g128, w1bd, b1bd, w2p, b2p, s2):
    return pl.pallas_call(
        _msg_body,
        grid=(EP // 8 // MB,),
        in_specs=[
            pl.BlockSpec((MB, 32), lambda i: (i, 0)),
            pl.BlockSpec((MB, 128), lambda i: (i, 0)),
            pl.BlockSpec((32, 1024), lambda i: (0, 0)),
            pl.BlockSpec((1, 1024), lambda i: (0, 0)),
            pl.BlockSpec((128, H * H), lambda i: (0, 0)),
            pl.BlockSpec((1, H * H), lambda i: (0, 0)),
            pl.BlockSpec((H * H, H), lambda i: (0, 0)),
        ],
        out_specs=pl.BlockSpec((MB, 128), lambda i: (i, 0)),
        out_shape=jax.ShapeDtypeStruct((EP // 8, 128), f32),
    )(ea32, g128, w1bd, b1bd, w2p, b2p, s2)


def _node_body(s_ref, a0_ref, a1_ref, wroot_ref, bconv_ref,
               wir_ref, wiz_ref, win_ref, bi_ref,
               whr_ref, whz_ref, whn_ref, bh_ref, o_ref):
    sv = s_ref[...]
    agg = a0_ref[...] + a1_ref[...]
    m = jnp.maximum(
        jnp.dot(sv, wroot_ref[...], preferred_element_type=f32) + agg
        + bconv_ref[...], 0.0)
    bi = bi_ref[...]
    bh = bh_ref[...]
    gir = jnp.dot(m, wir_ref[...], preferred_element_type=f32) + bi[:, :H]
    giz = jnp.dot(m, wiz_ref[...], preferred_element_type=f32) + bi[:, H:2 * H]
    gin = jnp.dot(m, win_ref[...], preferred_element_type=f32) + bi[:, 2 * H:]
    ghr = jnp.dot(sv, whr_ref[...], preferred_element_type=f32) + bh[:, :H]
    ghz = jnp.dot(sv, whz_ref[...], preferred_element_type=f32) + bh[:, H:2 * H]
    ghn = jnp.dot(sv, whn_ref[...], preferred_element_type=f32) + bh[:, 2 * H:]
    r = jax.nn.sigmoid(gir + ghr)
    z = jax.nn.sigmoid(giz + ghz)
    n = jnp.tanh(gin + r * ghn)
    o_ref[...] = (1.0 - z) * n + z * sv


def _node(s, agg0, agg1, wroot_t, bconv, wir, wiz, win, bi, whr, whz, whn, bh):
    wspec = pl.BlockSpec((H, H), lambda i: (0, 0))
    bspec = pl.BlockSpec((1, 3 * H), lambda i: (0, 0))
    nspec = pl.BlockSpec((BLK, H), lambda i: (i, 0))
    return pl.pallas_call(
        _node_body,
        grid=(NBLK,),
        in_specs=[
            nspec, nspec, nspec,
            wspec, pl.BlockSpec((1, H), lambda i: (0, 0)),
            wspec, wspec, wspec, bspec,
            wspec, wspec, wspec, bspec,
        ],
        out_specs=nspec,
        out_shape=jax.ShapeDtypeStruct((NP, H), f32),
    )(s, agg0, agg1, wroot_t, bconv, wir, wiz, win, bi, whr, whz, whn, bh)


def _epi_body(s_ref, x_ref, bat_ref, w1_ref, b1_ref, w2_ref, b2_ref,
              fg1_ref, fg2_ref, seg1_ref, seg2_ref, cnt_ref):
    i = pl.program_id(0)
    hmid = jnp.maximum(
        jnp.dot(s_ref[...], w1_ref[...], preferred_element_type=f32)
        + b1_ref[...], 0.0)
    emb = jnp.dot(hmid, w2_ref[...], preferred_element_type=f32) + b2_ref[...]
    xv = x_ref[...]
    ss = (jnp.sum(emb * emb, axis=1, keepdims=True)
          + jnp.sum(xv * xv, axis=1, keepdims=True))
    inv = 1.0 / jnp.maximum(jnp.sqrt(ss), 1e-12)
    fg1 = emb * inv
    fg2 = xv * inv
    fg1_ref[...] = fg1
    fg2_ref[...] = fg2
    onehot = (bat_ref[...] == lax.broadcasted_iota(jnp.int32, (BLK, B), 1)
              ).astype(f32)
    dn = (((0,), (0,)), ((), ()))
    p1 = lax.dot_general(onehot, fg1, dn, preferred_element_type=f32)
    p2 = lax.dot_general(onehot, fg2, dn, preferred_element_type=f32)
    pc = jnp.broadcast_to(jnp.sum(onehot, axis=0)[:, None], (B, IN))

    @pl.when(i == 0)
    def _():
        seg1_ref[...] = jnp.zeros_like(seg1_ref)
        seg2_ref[...] = jnp.zeros_like(seg2_ref)
        cnt_ref[...] = jnp.zeros_like(cnt_ref)

    seg1_ref[...] += p1
    seg2_ref[...] += p2
    cnt_ref[...] += pc


def _epilogue(s, x, bat2, w1_t, b1, w2_t, b2):
    zspec = lambda shape: pl.BlockSpec(shape, lambda i: (0, 0))
    return pl.pallas_call(
        _epi_body,
        grid=(NBLK,),
        in_specs=[
            pl.BlockSpec((BLK, H), lambda i: (i, 0)),
            pl.BlockSpec((BLK, IN), lambda i: (i, 0)),
            pl.BlockSpec((BLK, 1), lambda i: (i, 0)),
            zspec((H, H)), zspec((1, H)), zspec((H, EMB)), zspec((1, EMB)),
        ],
        out_specs=[
            pl.BlockSpec((BLK, EMB), lambda i: (i, 0)),
            pl.BlockSpec((BLK, IN), lambda i: (i, 0)),
            zspec((B, EMB)), zspec((B, IN)), zspec((B, IN)),
        ],
        out_shape=[
            jax.ShapeDtypeStruct((NP, EMB), f32),
            jax.ShapeDtypeStruct((NP, IN), f32),
            jax.ShapeDtypeStruct((B, EMB), f32),
            jax.ShapeDtypeStruct((B, IN), f32),
            jax.ShapeDtypeStruct((B, IN), f32),
        ],
    )(s, x, bat2, w1_t, b1, w2_t, b2)


def _ratio_body(s1_ref, s2_ref, cnt_ref, wp1_ref, wp2_ref, bp_ref, o_ref):
    cnt = jnp.maximum(cnt_ref[...], 1.0)
    r1 = s1_ref[...] / cnt[:, :EMB]
    r2 = s2_ref[...] / cnt
    v = (jnp.dot(r1, wp1_ref[...], preferred_element_type=f32)
         + jnp.dot(r2, wp2_ref[...], preferred_element_type=f32)
         + bp_ref[...])
    o_ref[...] = jax.nn.sigmoid(v)


def _ratio(seg1, seg2, cnt, wp1_t, wp2_t, bp):
    return pl.pallas_call(
        _ratio_body,
        out_shape=jax.ShapeDtypeStruct((B, 1), f32),
    )(seg1, seg2, cnt, wp1_t, wp2_t, bp)


def kernel(x, edge_attr, W_in_fc, b_in_fc, W_e1, b_e1, W_e2, b_e2,
           W_root, b_conv, W_ih, W_hh, b_ih, b_hh,
           W_o1, b_o1, W_o2, b_o2, W_p, b_p,
           edge_index, batch):
    src = jnp.zeros((EP,), jnp.int32).at[:E].set(edge_index[0])
    dst = jnp.zeros((EP,), jnp.int32).at[:E].set(edge_index[1])
    src3 = src.reshape(NW, NCH, CH)
    dst3 = dst.reshape(NW, NCH, CH)
    ea32 = jnp.zeros((EP // 8, 32), f32).at[:E // 8].set(
        edge_attr.reshape(E // 8, 32))
    zero_n = jnp.zeros((NP, H), f32)
    x_p = jnp.zeros((NP, IN), f32).at[:N].set(x)
    bat2 = jnp.full((NP, 1), B, jnp.int32).at[:N, 0].set(batch)

    s2 = jnp.repeat(jnp.eye(H, dtype=f32), H, axis=0)

    w_in_t = W_in_fc.T
    w1bd = jax.scipy.linalg.block_diag(*([W_e1.T] * 8))
    b1bd = jnp.tile(b_e1.reshape(1, 128), (1, 8))
    w2p = W_e2.T.reshape(128, H, H).transpose(0, 2, 1).reshape(128, H * H)
    b2p = b_e2.reshape(H, H).T.reshape(1, H * H)
    wroot_t = W_root.T
    wir, wiz, win = (W_ih[:H].T, W_ih[H:2 * H].T, W_ih[2 * H:].T)
    whr, whz, whn = (W_hh[:H].T, W_hh[H:2 * H].T, W_hh[2 * H:].T)
    wo1_t = W_o1.T
    wo2_t = W_o2.T
    wp1_t = W_p[:, :EMB].T
    wp2_t = W_p[:, EMB:].T

    b_in = b_in_fc.reshape(1, H)
    bconv = b_conv.reshape(1, H)
    bi = b_ih.reshape(1, 3 * H)
    bh = b_hh.reshape(1, 3 * H)
    bo1 = b_o1.reshape(1, H)
    bo2 = b_o2.reshape(1, EMB)
    bp = b_p.reshape(1, 1)

    s = _in_fc(x_p, w_in_t, b_in)
    for _ in range(ITERS):
        g = _sc_gather(s, src3)
        msg128 = _msg(ea32, g.reshape(EP // 8, 128), w1bd, b1bd, w2p, b2p,
                      s2)
        agg2 = _sc_scatter(msg128.reshape(EP, H), dst3, zero_n)
        s = _node(s, agg2[0], agg2[1], wroot_t, bconv, wir, wiz, win, bi,
                  whr, whz, whn, bh)
    fg1, fg2, seg1, seg2, cnt = _epilogue(s, x_p, bat2, wo1_t, bo1, wo2_t, bo2)
    fg_embed = jnp.concatenate([fg1[:N], fg2[:N]], axis=1)
    cg_fg_ratio = _ratio(seg1, seg2, cnt, wp1_t, wp2_t, bp)
    return (fg_embed, cg_fg_ratio)


# speedup vs baseline: 1.8179x; 1.0018x over previous
import functools

import jax
import jax.numpy as jnp
from jax import lax
from jax.experimental import pallas as pl
from jax.experimental.pallas import tpu as pltpu
from jax.experimental.pallas import tpu_sc as plsc

N = 10000
NP = 10240
E = 160000
IN = 128
H = 16
EMB = 64
B = 64
ITERS = 3

NC = 2
NS = 16
NW = NC * NS
CH = 128
NCH = 40
EPW = CH * NCH
EP = NW * EPW

MB = 512
NBLK = 10
BLK = NP // NBLK
RPS = NP // NS

f32 = jnp.float32


def _gather_body(s_hbm, src_hbm, g_hbm, idx_v, rows_v, tab, sem):
    c = lax.axis_index("c")
    s = lax.axis_index("s")
    wid = s * NC + c
    pltpu.sync_copy(s_hbm.at[pl.ds(s * RPS, RPS)], tab.at[pl.ds(s * RPS, RPS)])
    pltpu.sync_copy(src_hbm.at[wid], idx_v)
    plsc.subcore_barrier()

    def chunk(jo, carry):
        descs = []
        for ji in range(8):
            j = jo * 8 + ji
            descs.append(pltpu.async_copy(
                tab.at[idx_v.at[j]], rows_v.at[pl.ds(j * CH, CH)], sem))
        for d in descs:
            d.wait()
        return carry

    lax.fori_loop(0, NCH // 8, chunk, 0)
    pltpu.sync_copy(rows_v, g_hbm.at[pl.ds(wid * EPW, EPW)])


@functools.cache
def _sc_gather_kernel():
    mesh = plsc.VectorSubcoreMesh(core_axis_name="c", subcore_axis_name="s",
                                  num_cores=NC, num_subcores=NS)
    return functools.partial(
        pl.kernel,
        mesh=mesh,
        compiler_params=pltpu.CompilerParams(use_tc_tiling_on_sc=False),
        out_type=jax.ShapeDtypeStruct((EP, H), f32),
        scratch_types=[
            pltpu.VMEM((NCH, CH), jnp.int32),
            pltpu.VMEM((EPW, H), f32),
            pltpu.VMEM_SHARED((NP, H), f32),
            pltpu.SemaphoreType.DMA,
        ],
    )(_gather_body)


def _sc_gather(s, src3):
    return _sc_gather_kernel()(s, src3)


def _scatter_body(msg_hbm, dst_hbm, zero_hbm, agg_hbm, idx_v, rows_v, acc,
                  sem):
    c = lax.axis_index("c")
    s = lax.axis_index("s")
    wid = s * NC + c
    pltpu.sync_copy(zero_hbm.at[pl.ds(s * RPS, RPS)], acc.at[pl.ds(s * RPS, RPS)])
    pltpu.sync_copy(dst_hbm.at[wid], idx_v)
    pltpu.sync_copy(msg_hbm.at[pl.ds(wid * EPW, EPW)], rows_v)
    plsc.subcore_barrier()

    def chunk(jo, carry):
        descs = []
        for ji in range(8):
            j = jo * 8 + ji
            descs.append(pltpu.async_copy(
                rows_v.at[pl.ds(j * CH, CH)], acc.at[idx_v.at[j]], sem,
                add=True))
        for d in descs:
            d.wait()
        return carry

    lax.fori_loop(0, NCH // 8, chunk, 0)
    plsc.subcore_barrier()
    pltpu.sync_copy(acc.at[pl.ds(s * RPS, RPS)], agg_hbm.at[c, pl.ds(s * RPS, RPS)])


@functools.cache
def _sc_scatter_kernel():
    mesh = plsc.VectorSubcoreMesh(core_axis_name="c", subcore_axis_name="s",
                                  num_cores=NC, num_subcores=NS)
    return functools.partial(
        pl.kernel,
        mesh=mesh,
        compiler_params=pltpu.CompilerParams(use_tc_tiling_on_sc=False),
        out_type=jax.ShapeDtypeStruct((NC, NP, H), f32),
        scratch_types=[
            pltpu.VMEM((NCH, CH), jnp.int32),
            pltpu.VMEM((EPW, H), f32),
            pltpu.VMEM_SHARED((NP, H), f32),
            pltpu.SemaphoreType.DMA,
        ],
    )(_scatter_body)


def _sc_scatter(msg, dst3, zero_n):
    return _sc_scatter_kernel()(msg, dst3, zero_n)


def _in_fc_body(x_ref, w_ref, b_ref, o_ref):
    o_ref[...] = jnp.maximum(
        jnp.dot(x_ref[...], w_ref[...], preferred_element_type=f32)
        + b_ref[...], 0.0)


def _in_fc(x, w_t, b):
    return pl.pallas_call(
        _in_fc_body,
        grid=(NBLK,),
        in_specs=[
            pl.BlockSpec((BLK, IN), lambda i: (i, 0)),
            pl.BlockSpec((IN, H), lambda i: (0, 0)),
            pl.BlockSpec((1, H), lambda i: (0, 0)),
        ],
        out_specs=pl.BlockSpec((BLK, H), lambda i: (i, 0)),
        out_shape=jax.ShapeDtypeStruct((NP, H), f32),
    )(x, w_t, b)


def _msg_body(ea_ref, g_ref, w1_ref, b1_ref, w2_ref, b2_ref, s_ref,
              msg_ref):
    t = pl.program_id(0)
    bf = jnp.bfloat16
    rmask = (t * MB + lax.broadcasted_iota(jnp.int32, (MB, 1), 0)) < E // 8
    abd = jnp.maximum(
        jnp.dot(ea_ref[...].astype(bf), w1_ref[...].astype(bf),
                preferred_element_type=f32) + b1_ref[...], 0.0).astype(bf)
    w2 = w2_ref[...].astype(bf)
    s_m = s_ref[...].astype(bf)
    gbf = g_ref[...].astype(bf)
    for p in range(8):
        we = (jnp.dot(abd[:, 128 * p:128 * p + 128], w2,
                      preferred_element_type=f32) + b2_ref[...]).astype(bf)
        grep = jnp.tile(gbf[:, H * p:H * p + H], (1, H))
        msg = jnp.dot(grep * we, s_m, preferred_element_type=f32)
        msg_ref[:, H * p:H * p + H] = jnp.where(rmask, msg, 0.0)


def _msg(ea32, g128, w1bd, b1bd, w2p, b2p, s2):
    return pl.pallas_call(
        _msg_body,
        grid=(EP // 8 // MB,),
        in_specs=[
            pl.BlockSpec((MB, 32), lambda i: (i, 0)),
            pl.BlockSpec((MB, 128), lambda i: (i, 0)),
            pl.BlockSpec((32, 1024), lambda i: (0, 0)),
            pl.BlockSpec((1, 1024), lambda i: (0, 0)),
            pl.BlockSpec((128, H * H), lambda i: (0, 0)),
            pl.BlockSpec((1, H * H), lambda i: (0, 0)),
            pl.BlockSpec((H * H, H), lambda i: (0, 0)),
        ],
        out_specs=pl.BlockSpec((MB, 128), lambda i: (i, 0)),
        out_shape=jax.ShapeDtypeStruct((EP // 8, 128), f32),
    )(ea32, g128, w1bd, b1bd, w2p, b2p, s2)


def _node_body(s_ref, a0_ref, a1_ref, wroot_ref, bconv_ref,
               wir_ref, wiz_ref, win_ref, bi_ref,
               whr_ref, whz_ref, whn_ref, bh_ref, o_ref):
    sv = s_ref[...]
    agg = a0_ref[...] + a1_ref[...]
    m = jnp.maximum(
        jnp.dot(sv, wroot_ref[...], preferred_element_type=f32) + agg
        + bconv_ref[...], 0.0)
    bi = bi_ref[...]
    bh = bh_ref[...]
    gir = jnp.dot(m, wir_ref[...], preferred_element_type=f32) + bi[:, :H]
    giz = jnp.dot(m, wiz_ref[...], preferred_element_type=f32) + bi[:, H:2 * H]
    gin = jnp.dot(m, win_ref[...], preferred_element_type=f32) + bi[:, 2 * H:]
    ghr = jnp.dot(sv, whr_ref[...], preferred_element_type=f32) + bh[:, :H]
    ghz = jnp.dot(sv, whz_ref[...], preferred_element_type=f32) + bh[:, H:2 * H]
    ghn = jnp.dot(sv, whn_ref[...], preferred_element_type=f32) + bh[:, 2 * H:]
    r = jax.nn.sigmoid(gir + ghr)
    z = jax.nn.sigmoid(giz + ghz)
    n = jnp.tanh(gin + r * ghn)
    o_ref[...] = (1.0 - z) * n + z * sv


def _node(s, agg0, agg1, wroot_t, bconv, wir, wiz, win, bi, whr, whz, whn, bh):
    wspec = pl.BlockSpec((H, H), lambda i: (0, 0))
    bspec = pl.BlockSpec((1, 3 * H), lambda i: (0, 0))
    nspec = pl.BlockSpec((BLK, H), lambda i: (i, 0))
    return pl.pallas_call(
        _node_body,
        grid=(NBLK,),
        in_specs=[
            nspec, nspec, nspec,
            wspec, pl.BlockSpec((1, H), lambda i: (0, 0)),
            wspec, wspec, wspec, bspec,
            wspec, wspec, wspec, bspec,
        ],
        out_specs=nspec,
        out_shape=jax.ShapeDtypeStruct((NP, H), f32),
    )(s, agg0, agg1, wroot_t, bconv, wir, wiz, win, bi, whr, whz, whn, bh)


def _epi_body(s_ref, x_ref, bat_ref, w1_ref, b1_ref, w2_ref, b2_ref,
              fg1_ref, fg2_ref, seg1_ref, seg2_ref, cnt_ref):
    i = pl.program_id(0)
    hmid = jnp.maximum(
        jnp.dot(s_ref[...], w1_ref[...], preferred_element_type=f32)
        + b1_ref[...], 0.0)
    emb = jnp.dot(hmid, w2_ref[...], preferred_element_type=f32) + b2_ref[...]
    xv = x_ref[...]
    ss = (jnp.sum(emb * emb, axis=1, keepdims=True)
          + jnp.sum(xv * xv, axis=1, keepdims=True))
    inv = 1.0 / jnp.maximum(jnp.sqrt(ss), 1e-12)
    fg1 = emb * inv
    fg2 = xv * inv
    fg1_ref[...] = fg1
    fg2_ref[...] = fg2
    onehot = (bat_ref[...] == lax.broadcasted_iota(jnp.int32, (BLK, B), 1)
              ).astype(f32)
    dn = (((0,), (0,)), ((), ()))
    p1 = lax.dot_general(onehot, fg1, dn, preferred_element_type=f32)
    p2 = lax.dot_general(onehot, fg2, dn, preferred_element_type=f32)
    pc = jnp.broadcast_to(jnp.sum(onehot, axis=0)[:, None], (B, IN))

    @pl.when(i == 0)
    def _():
        seg1_ref[...] = jnp.zeros_like(seg1_ref)
        seg2_ref[...] = jnp.zeros_like(seg2_ref)
        cnt_ref[...] = jnp.zeros_like(cnt_ref)

    seg1_ref[...] += p1
    seg2_ref[...] += p2
    cnt_ref[...] += pc


def _epilogue(s, x, bat2, w1_t, b1, w2_t, b2):
    zspec = lambda shape: pl.BlockSpec(shape, lambda i: (0, 0))
    return pl.pallas_call(
        _epi_body,
        grid=(NBLK,),
        in_specs=[
            pl.BlockSpec((BLK, H), lambda i: (i, 0)),
            pl.BlockSpec((BLK, IN), lambda i: (i, 0)),
            pl.BlockSpec((BLK, 1), lambda i: (i, 0)),
            zspec((H, H)), zspec((1, H)), zspec((H, EMB)), zspec((1, EMB)),
        ],
        out_specs=[
            pl.BlockSpec((BLK, EMB), lambda i: (i, 0)),
            pl.BlockSpec((BLK, IN), lambda i: (i, 0)),
            zspec((B, EMB)), zspec((B, IN)), zspec((B, IN)),
        ],
        out_shape=[
            jax.ShapeDtypeStruct((NP, EMB), f32),
            jax.ShapeDtypeStruct((NP, IN), f32),
            jax.ShapeDtypeStruct((B, EMB), f32),
            jax.ShapeDtypeStruct((B, IN), f32),
            jax.ShapeDtypeStruct((B, IN), f32),
        ],
    )(s, x, bat2, w1_t, b1, w2_t, b2)


def _ratio_body(s1_ref, s2_ref, cnt_ref, wp1_ref, wp2_ref, bp_ref, o_ref):
    cnt = jnp.maximum(cnt_ref[...], 1.0)
    r1 = s1_ref[...] / cnt[:, :EMB]
    r2 = s2_ref[...] / cnt
    v = (jnp.dot(r1, wp1_ref[...], preferred_element_type=f32)
         + jnp.dot(r2, wp2_ref[...], preferred_element_type=f32)
         + bp_ref[...])
    o_ref[...] = jax.nn.sigmoid(v)


def _ratio(seg1, seg2, cnt, wp1_t, wp2_t, bp):
    return pl.pallas_call(
        _ratio_body,
        out_shape=jax.ShapeDtypeStruct((B, 1), f32),
    )(seg1, seg2, cnt, wp1_t, wp2_t, bp)


def kernel(x, edge_attr, W_in_fc, b_in_fc, W_e1, b_e1, W_e2, b_e2,
           W_root, b_conv, W_ih, W_hh, b_ih, b_hh,
           W_o1, b_o1, W_o2, b_o2, W_p, b_p,
           edge_index, batch):
    src = jnp.zeros((EP,), jnp.int32).at[:E].set(edge_index[0])
    dst = jnp.zeros((EP,), jnp.int32).at[:E].set(edge_index[1])
    src3 = src.reshape(NW, NCH, CH)
    dst3 = dst.reshape(NW, NCH, CH)
    ea32 = jnp.zeros((EP // 8, 32), f32).at[:E // 8].set(
        edge_attr.reshape(E // 8, 32))
    zero_n = jnp.zeros((NP, H), f32)
    x_p = jnp.zeros((NP, IN), f32).at[:N].set(x)
    bat2 = jnp.full((NP, 1), B, jnp.int32).at[:N, 0].set(batch)

    s2 = jnp.repeat(jnp.eye(H, dtype=f32), H, axis=0)

    w_in_t = W_in_fc.T
    w1bd = jax.scipy.linalg.block_diag(*([W_e1.T] * 8))
    b1bd = jnp.tile(b_e1.reshape(1, 128), (1, 8))
    w2p = W_e2.T.reshape(128, H, H).transpose(0, 2, 1).reshape(128, H * H)
    b2p = b_e2.reshape(H, H).T.reshape(1, H * H)
    wroot_t = W_root.T
    wir, wiz, win = (W_ih[:H].T, W_ih[H:2 * H].T, W_ih[2 * H:].T)
    whr, whz, whn = (W_hh[:H].T, W_hh[H:2 * H].T, W_hh[2 * H:].T)
    wo1_t = W_o1.T
    wo2_t = W_o2.T
    wp1_t = W_p[:, :EMB].T
    wp2_t = W_p[:, EMB:].T

    b_in = b_in_fc.reshape(1, H)
    bconv = b_conv.reshape(1, H)
    bi = b_ih.reshape(1, 3 * H)
    bh = b_hh.reshape(1, 3 * H)
    bo1 = b_o1.reshape(1, H)
    bo2 = b_o2.reshape(1, EMB)
    bp = b_p.reshape(1, 1)

    s = _in_fc(x_p, w_in_t, b_in)
    for _ in range(ITERS):
        g = _sc_gather(s, src3)
        msg128 = _msg(ea32, g.reshape(EP // 8, 128), w1bd, b1bd, w2p, b2p,
                      s2)
        agg2 = _sc_scatter(msg128.reshape(EP, H), dst3, zero_n)
        s = _node(s, agg2[0], agg2[1], wroot_t, bconv, wir, wiz, win, bi,
                  whr, whz, whn, bh)
    fg1, fg2, seg1, seg2, cnt = _epilogue(s, x_p, bat2, wo1_t, bo1, wo2_t, bo2)
    fg_embed = jnp.concatenate([fg1[:N], fg2[:N]], axis=1)
    cg_fg_ratio = _ratio(seg1, seg2, cnt, wp1_t, wp2_t, bp)
    return (fg_embed, cg_fg_ratio)


# speedup vs baseline: 1.8350x; 1.0094x over previous
import functools

import jax
import jax.numpy as jnp
from jax import lax
from jax.experimental import pallas as pl
from jax.experimental.pallas import tpu as pltpu
from jax.experimental.pallas import tpu_sc as plsc

N = 10000
NP = 10240
E = 160000
IN = 128
H = 16
EMB = 64
B = 64
ITERS = 3

NC = 2
NS = 16
NW = NC * NS
CH = 128
NCH = 40
EPW = CH * NCH
EP = NW * EPW

MB = 512
NBLK = 10
BLK = NP // NBLK
RPS = NP // NS

f32 = jnp.float32


def _gather_body(s_hbm, src_hbm, g_hbm, idx_v, rows_v, tab, sem):
    c = lax.axis_index("c")
    s = lax.axis_index("s")
    wid = s * NC + c
    pltpu.sync_copy(s_hbm.at[pl.ds(s * RPS, RPS)], tab.at[pl.ds(s * RPS, RPS)])
    pltpu.sync_copy(src_hbm.at[wid], idx_v)
    plsc.subcore_barrier()

    def chunk(jo, carry):
        descs = []
        for ji in range(8):
            j = jo * 8 + ji
            descs.append(pltpu.async_copy(
                tab.at[idx_v.at[j]], rows_v.at[pl.ds(j * CH, CH)], sem))
        for d in descs:
            d.wait()
        return carry

    lax.fori_loop(0, NCH // 8, chunk, 0)
    pltpu.sync_copy(rows_v, g_hbm.at[pl.ds(wid * EPW, EPW)])


@functools.cache
def _sc_gather_kernel():
    mesh = plsc.VectorSubcoreMesh(core_axis_name="c", subcore_axis_name="s",
                                  num_cores=NC, num_subcores=NS)
    return functools.partial(
        pl.kernel,
        mesh=mesh,
        compiler_params=pltpu.CompilerParams(use_tc_tiling_on_sc=False),
        out_type=jax.ShapeDtypeStruct((EP, H), f32),
        scratch_types=[
            pltpu.VMEM((NCH, CH), jnp.int32),
            pltpu.VMEM((EPW, H), f32),
            pltpu.VMEM_SHARED((NP, H), f32),
            pltpu.SemaphoreType.DMA,
        ],
    )(_gather_body)


def _sc_gather(s, src3):
    return _sc_gather_kernel()(s, src3)


def _scatter_body(msg_hbm, dst_hbm, zero_hbm, agg_hbm, idx_v, rows_v, acc,
                  sem):
    c = lax.axis_index("c")
    s = lax.axis_index("s")
    wid = s * NC + c
    pltpu.sync_copy(zero_hbm.at[pl.ds(s * RPS, RPS)], acc.at[pl.ds(s * RPS, RPS)])
    pltpu.sync_copy(dst_hbm.at[wid], idx_v)
    pltpu.sync_copy(msg_hbm.at[pl.ds(wid * EPW, EPW)], rows_v)
    plsc.subcore_barrier()

    def chunk(jo, carry):
        descs = []
        for ji in range(8):
            j = jo * 8 + ji
            descs.append(pltpu.async_copy(
                rows_v.at[pl.ds(j * CH, CH)], acc.at[idx_v.at[j]], sem,
                add=True))
        for d in descs:
            d.wait()
        return carry

    lax.fori_loop(0, NCH // 8, chunk, 0)
    plsc.subcore_barrier()
    pltpu.sync_copy(acc.at[pl.ds(s * RPS, RPS)], agg_hbm.at[c, pl.ds(s * RPS, RPS)])


@functools.cache
def _sc_scatter_kernel():
    mesh = plsc.VectorSubcoreMesh(core_axis_name="c", subcore_axis_name="s",
                                  num_cores=NC, num_subcores=NS)
    return functools.partial(
        pl.kernel,
        mesh=mesh,
        compiler_params=pltpu.CompilerParams(use_tc_tiling_on_sc=False),
        out_type=jax.ShapeDtypeStruct((NC, NP, H), f32),
        scratch_types=[
            pltpu.VMEM((NCH, CH), jnp.int32),
            pltpu.VMEM((EPW, H), f32),
            pltpu.VMEM_SHARED((NP, H), f32),
            pltpu.SemaphoreType.DMA,
        ],
    )(_scatter_body)


def _sc_scatter(msg, dst3, zero_n):
    return _sc_scatter_kernel()(msg, dst3, zero_n)


def _in_fc_body(x_ref, w_ref, b_ref, o_ref):
    o_ref[...] = jnp.maximum(
        jnp.dot(x_ref[...], w_ref[...], preferred_element_type=f32)
        + b_ref[...], 0.0)


def _in_fc(x, w_t, b):
    return pl.pallas_call(
        _in_fc_body,
        grid=(NBLK,),
        in_specs=[
            pl.BlockSpec((BLK, IN), lambda i: (i, 0)),
            pl.BlockSpec((IN, H), lambda i: (0, 0)),
            pl.BlockSpec((1, H), lambda i: (0, 0)),
        ],
        out_specs=pl.BlockSpec((BLK, H), lambda i: (i, 0)),
        out_shape=jax.ShapeDtypeStruct((NP, H), f32),
    )(x, w_t, b)


def _msg_body(ea_ref, g_ref, w1_ref, b1_ref, w2_ref, b2_ref, s_ref,
              msg_ref):
    t = pl.program_id(0)
    bf = jnp.bfloat16
    rmask = (t * MB + lax.broadcasted_iota(jnp.int32, (MB, 1), 0)) < E // 8
    abd = jnp.maximum(
        jnp.dot(ea_ref[...].astype(bf), w1_ref[...].astype(bf),
                preferred_element_type=f32) + b1_ref[...], 0.0).astype(bf)
    w2 = w2_ref[...].astype(bf)
    s_m = s_ref[...].astype(bf)
    b2 = b2_ref[...].astype(bf)
    gbf = g_ref[...].astype(bf)
    wes = [jnp.dot(abd[:, 128 * p:128 * p + 128], w2,
                   preferred_element_type=f32).astype(bf) for p in range(8)]
    prods = [jnp.tile(gbf[:, H * p:H * p + H], (1, H)) * (wes[p] + b2)
             for p in range(8)]
    for p in range(8):
        msg = jnp.dot(prods[p], s_m, preferred_element_type=f32)
        msg_ref[:, H * p:H * p + H] = jnp.where(rmask, msg, 0.0)


def _msg(ea32, g128, w1bd, b1bd, w2p, b2p, s2):
    return pl.pallas_call(
        _msg_body,
        grid=(EP // 8 // MB,),
        in_specs=[
            pl.BlockSpec((MB, 32), lambda i: (i, 0)),
            pl.BlockSpec((MB, 128), lambda i: (i, 0)),
            pl.BlockSpec((32, 1024), lambda i: (0, 0)),
            pl.BlockSpec((1, 1024), lambda i: (0, 0)),
            pl.BlockSpec((128, H * H), lambda i: (0, 0)),
            pl.BlockSpec((1, H * H), lambda i: (0, 0)),
            pl.BlockSpec((H * H, H), lambda i: (0, 0)),
        ],
        out_specs=pl.BlockSpec((MB, 128), lambda i: (i, 0)),
        out_shape=jax.ShapeDtypeStruct((EP // 8, 128), f32),
    )(ea32, g128, w1bd, b1bd, w2p, b2p, s2)


def _node_body(s_ref, a0_ref, a1_ref, wroot_ref, bconv_ref,
               wir_ref, wiz_ref, win_ref, bi_ref,
               whr_ref, whz_ref, whn_ref, bh_ref, o_ref):
    sv = s_ref[...]
    agg = a0_ref[...] + a1_ref[...]
    m = jnp.maximum(
        jnp.dot(sv, wroot_ref[...], preferred_element_type=f32) + agg
        + bconv_ref[...], 0.0)
    bi = bi_ref[...]
    bh = bh_ref[...]
    gir = jnp.dot(m, wir_ref[...], preferred_element_type=f32) + bi[:, :H]
    giz = jnp.dot(m, wiz_ref[...], preferred_element_type=f32) + bi[:, H:2 * H]
    gin = jnp.dot(m, win_ref[...], preferred_element_type=f32) + bi[:, 2 * H:]
    ghr = jnp.dot(sv, whr_ref[...], preferred_element_type=f32) + bh[:, :H]
    ghz = jnp.dot(sv, whz_ref[...], preferred_element_type=f32) + bh[:, H:2 * H]
    ghn = jnp.dot(sv, whn_ref[...], preferred_element_type=f32) + bh[:, 2 * H:]
    r = jax.nn.sigmoid(gir + ghr)
    z = jax.nn.sigmoid(giz + ghz)
    n = jnp.tanh(gin + r * ghn)
    o_ref[...] = (1.0 - z) * n + z * sv


def _node(s, agg0, agg1, wroot_t, bconv, wir, wiz, win, bi, whr, whz, whn, bh):
    wspec = pl.BlockSpec((H, H), lambda i: (0, 0))
    bspec = pl.BlockSpec((1, 3 * H), lambda i: (0, 0))
    nspec = pl.BlockSpec((BLK, H), lambda i: (i, 0))
    return pl.pallas_call(
        _node_body,
        grid=(NBLK,),
        in_specs=[
            nspec, nspec, nspec,
            wspec, pl.BlockSpec((1, H), lambda i: (0, 0)),
            wspec, wspec, wspec, bspec,
            wspec, wspec, wspec, bspec,
        ],
        out_specs=nspec,
        out_shape=jax.ShapeDtypeStruct((NP, H), f32),
    )(s, agg0, agg1, wroot_t, bconv, wir, wiz, win, bi, whr, whz, whn, bh)


def _epi_body(s_ref, x_ref, bat_ref, w1_ref, b1_ref, w2_ref, b2_ref,
              fg1_ref, fg2_ref, seg1_ref, seg2_ref, cnt_ref):
    i = pl.program_id(0)
    hmid = jnp.maximum(
        jnp.dot(s_ref[...], w1_ref[...], preferred_element_type=f32)
        + b1_ref[...], 0.0)
    emb = jnp.dot(hmid, w2_ref[...], preferred_element_type=f32) + b2_ref[...]
    xv = x_ref[...]
    ss = (jnp.sum(emb * emb, axis=1, keepdims=True)
          + jnp.sum(xv * xv, axis=1, keepdims=True))
    inv = 1.0 / jnp.maximum(jnp.sqrt(ss), 1e-12)
    fg1 = emb * inv
    fg2 = xv * inv
    fg1_ref[...] = fg1
    fg2_ref[...] = fg2
    onehot = (bat_ref[...] == lax.broadcasted_iota(jnp.int32, (BLK, B), 1)
              ).astype(f32)
    dn = (((0,), (0,)), ((), ()))
    p1 = lax.dot_general(onehot, fg1, dn, preferred_element_type=f32)
    p2 = lax.dot_general(onehot, fg2, dn, preferred_element_type=f32)
    pc = jnp.broadcast_to(jnp.sum(onehot, axis=0)[:, None], (B, IN))

    @pl.when(i == 0)
    def _():
        seg1_ref[...] = jnp.zeros_like(seg1_ref)
        seg2_ref[...] = jnp.zeros_like(seg2_ref)
        cnt_ref[...] = jnp.zeros_like(cnt_ref)

    seg1_ref[...] += p1
    seg2_ref[...] += p2
    cnt_ref[...] += pc


def _epilogue(s, x, bat2, w1_t, b1, w2_t, b2):
    zspec = lambda shape: pl.BlockSpec(shape, lambda i: (0, 0))
    return pl.pallas_call(
        _epi_body,
        grid=(NBLK,),
        in_specs=[
            pl.BlockSpec((BLK, H), lambda i: (i, 0)),
            pl.BlockSpec((BLK, IN), lambda i: (i, 0)),
            pl.BlockSpec((BLK, 1), lambda i: (i, 0)),
            zspec((H, H)), zspec((1, H)), zspec((H, EMB)), zspec((1, EMB)),
        ],
        out_specs=[
            pl.BlockSpec((BLK, EMB), lambda i: (i, 0)),
            pl.BlockSpec((BLK, IN), lambda i: (i, 0)),
            zspec((B, EMB)), zspec((B, IN)), zspec((B, IN)),
        ],
        out_shape=[
            jax.ShapeDtypeStruct((NP, EMB), f32),
            jax.ShapeDtypeStruct((NP, IN), f32),
            jax.ShapeDtypeStruct((B, EMB), f32),
            jax.ShapeDtypeStruct((B, IN), f32),
            jax.ShapeDtypeStruct((B, IN), f32),
        ],
    )(s, x, bat2, w1_t, b1, w2_t, b2)


def _ratio_body(s1_ref, s2_ref, cnt_ref, wp1_ref, wp2_ref, bp_ref, o_ref):
    cnt = jnp.maximum(cnt_ref[...], 1.0)
    r1 = s1_ref[...] / cnt[:, :EMB]
    r2 = s2_ref[...] / cnt
    v = (jnp.dot(r1, wp1_ref[...], preferred_element_type=f32)
         + jnp.dot(r2, wp2_ref[...], preferred_element_type=f32)
         + bp_ref[...])
    o_ref[...] = jax.nn.sigmoid(v)


def _ratio(seg1, seg2, cnt, wp1_t, wp2_t, bp):
    return pl.pallas_call(
        _ratio_body,
        out_shape=jax.ShapeDtypeStruct((B, 1), f32),
    )(seg1, seg2, cnt, wp1_t, wp2_t, bp)


def kernel(x, edge_attr, W_in_fc, b_in_fc, W_e1, b_e1, W_e2, b_e2,
           W_root, b_conv, W_ih, W_hh, b_ih, b_hh,
           W_o1, b_o1, W_o2, b_o2, W_p, b_p,
           edge_index, batch):
    src = jnp.zeros((EP,), jnp.int32).at[:E].set(edge_index[0])
    dst = jnp.zeros((EP,), jnp.int32).at[:E].set(edge_index[1])
    src3 = src.reshape(NW, NCH, CH)
    dst3 = dst.reshape(NW, NCH, CH)
    ea32 = jnp.zeros((EP // 8, 32), f32).at[:E // 8].set(
        edge_attr.reshape(E // 8, 32))
    zero_n = jnp.zeros((NP, H), f32)
    x_p = jnp.zeros((NP, IN), f32).at[:N].set(x)
    bat2 = jnp.full((NP, 1), B, jnp.int32).at[:N, 0].set(batch)

    s2 = jnp.repeat(jnp.eye(H, dtype=f32), H, axis=0)

    w_in_t = W_in_fc.T
    w1bd = jax.scipy.linalg.block_diag(*([W_e1.T] * 8))
    b1bd = jnp.tile(b_e1.reshape(1, 128), (1, 8))
    w2p = W_e2.T.reshape(128, H, H).transpose(0, 2, 1).reshape(128, H * H)
    b2p = b_e2.reshape(H, H).T.reshape(1, H * H)
    wroot_t = W_root.T
    wir, wiz, win = (W_ih[:H].T, W_ih[H:2 * H].T, W_ih[2 * H:].T)
    whr, whz, whn = (W_hh[:H].T, W_hh[H:2 * H].T, W_hh[2 * H:].T)
    wo1_t = W_o1.T
    wo2_t = W_o2.T
    wp1_t = W_p[:, :EMB].T
    wp2_t = W_p[:, EMB:].T

    b_in = b_in_fc.reshape(1, H)
    bconv = b_conv.reshape(1, H)
    bi = b_ih.reshape(1, 3 * H)
    bh = b_hh.reshape(1, 3 * H)
    bo1 = b_o1.reshape(1, H)
    bo2 = b_o2.reshape(1, EMB)
    bp = b_p.reshape(1, 1)

    s = _in_fc(x_p, w_in_t, b_in)
    for _ in range(ITERS):
        g = _sc_gather(s, src3)
        msg128 = _msg(ea32, g.reshape(EP // 8, 128), w1bd, b1bd, w2p, b2p,
                      s2)
        agg2 = _sc_scatter(msg128.reshape(EP, H), dst3, zero_n)
        s = _node(s, agg2[0], agg2[1], wroot_t, bconv, wir, wiz, win, bi,
                  whr, whz, whn, bh)
    fg1, fg2, seg1, seg2, cnt = _epilogue(s, x_p, bat2, wo1_t, bo1, wo2_t, bo2)
    fg_embed = jnp.concatenate([fg1[:N], fg2[:N]], axis=1)
    cg_fg_ratio = _ratio(seg1, seg2, cnt, wp1_t, wp2_t, bp)
    return (fg_embed, cg_fg_ratio)


# speedup vs baseline: 1.9034x; 1.0373x over previous
import functools

import jax
import jax.numpy as jnp
from jax import lax
from jax.experimental import pallas as pl
from jax.experimental.pallas import tpu as pltpu
from jax.experimental.pallas import tpu_sc as plsc

N = 10000
NP = 10240
E = 160000
IN = 128
H = 16
EMB = 64
B = 64
ITERS = 3

NC = 2
NS = 16
NW = NC * NS
CH = 128
NCH = 40
EPW = CH * NCH
EP = NW * EPW

MB = 512
NBLK = 10
BLK = NP // NBLK
RPS = NP // NS

f32 = jnp.float32


def _gather_body(s_hbm, src_hbm, g_hbm, idx_v, rows_v, tab, sem):
    c = lax.axis_index("c")
    s = lax.axis_index("s")
    wid = s * NC + c
    pltpu.sync_copy(s_hbm.at[pl.ds(s * RPS, RPS)], tab.at[pl.ds(s * RPS, RPS)])
    pltpu.sync_copy(src_hbm.at[wid], idx_v)
    plsc.subcore_barrier()

    def chunk(jo, carry):
        descs = []
        for ji in range(8):
            j = jo * 8 + ji
            descs.append(pltpu.async_copy(
                tab.at[idx_v.at[j]], rows_v.at[pl.ds(j * CH, CH)], sem))
        for d in descs:
            d.wait()
        return carry

    lax.fori_loop(0, NCH // 8, chunk, 0)
    pltpu.sync_copy(rows_v, g_hbm.at[pl.ds(wid * EPW, EPW)])


@functools.cache
def _sc_gather_kernel():
    mesh = plsc.VectorSubcoreMesh(core_axis_name="c", subcore_axis_name="s",
                                  num_cores=NC, num_subcores=NS)
    return functools.partial(
        pl.kernel,
        mesh=mesh,
        compiler_params=pltpu.CompilerParams(use_tc_tiling_on_sc=False),
        out_type=jax.ShapeDtypeStruct((EP, H), f32),
        scratch_types=[
            pltpu.VMEM((NCH, CH), jnp.int32),
            pltpu.VMEM((EPW, H), f32),
            pltpu.VMEM_SHARED((NP, H), f32),
            pltpu.SemaphoreType.DMA,
        ],
    )(_gather_body)


def _sc_gather(s, src3):
    return _sc_gather_kernel()(s, src3)


def _scatter_body(msg_hbm, dst_hbm, zero_hbm, agg_hbm, idx_v, rows_v, acc,
                  sem):
    c = lax.axis_index("c")
    s = lax.axis_index("s")
    wid = s * NC + c
    pltpu.sync_copy(zero_hbm.at[pl.ds(s * RPS, RPS)], acc.at[pl.ds(s * RPS, RPS)])
    pltpu.sync_copy(dst_hbm.at[wid], idx_v)
    pltpu.sync_copy(msg_hbm.at[pl.ds(wid * EPW, EPW)], rows_v)
    plsc.subcore_barrier()

    def chunk(jo, carry):
        descs = []
        for ji in range(8):
            j = jo * 8 + ji
            descs.append(pltpu.async_copy(
                rows_v.at[pl.ds(j * CH, CH)], acc.at[idx_v.at[j]], sem,
                add=True))
        for d in descs:
            d.wait()
        return carry

    lax.fori_loop(0, NCH // 8, chunk, 0)
    plsc.subcore_barrier()
    pltpu.sync_copy(acc.at[pl.ds(s * RPS, RPS)], agg_hbm.at[c, pl.ds(s * RPS, RPS)])


@functools.cache
def _sc_scatter_kernel():
    mesh = plsc.VectorSubcoreMesh(core_axis_name="c", subcore_axis_name="s",
                                  num_cores=NC, num_subcores=NS)
    return functools.partial(
        pl.kernel,
        mesh=mesh,
        compiler_params=pltpu.CompilerParams(use_tc_tiling_on_sc=False),
        out_type=jax.ShapeDtypeStruct((NC, NP, H), f32),
        scratch_types=[
            pltpu.VMEM((NCH, CH), jnp.int32),
            pltpu.VMEM((EPW, H), f32),
            pltpu.VMEM_SHARED((NP, H), f32),
            pltpu.SemaphoreType.DMA,
        ],
    )(_scatter_body)


def _sc_scatter(msg, dst3, zero_n):
    return _sc_scatter_kernel()(msg, dst3, zero_n)


def _in_fc_body(x_ref, w_ref, b_ref, o_ref):
    o_ref[...] = jnp.maximum(
        jnp.dot(x_ref[...], w_ref[...], preferred_element_type=f32)
        + b_ref[...], 0.0)


def _in_fc(x, w_t, b):
    return pl.pallas_call(
        _in_fc_body,
        grid=(NBLK,),
        in_specs=[
            pl.BlockSpec((BLK, IN), lambda i: (i, 0)),
            pl.BlockSpec((IN, H), lambda i: (0, 0)),
            pl.BlockSpec((1, H), lambda i: (0, 0)),
        ],
        out_specs=pl.BlockSpec((BLK, H), lambda i: (i, 0)),
        out_shape=jax.ShapeDtypeStruct((NP, H), f32),
    )(x, w_t, b)


def _msg_body(ea_ref, g_ref, w1_ref, b1_ref, w2_ref, b2_ref, s_ref,
              msg_ref):
    t = pl.program_id(0)
    bf = jnp.bfloat16
    rmask = (t * MB + lax.broadcasted_iota(jnp.int32, (MB, 1), 0)) < E // 8
    abd = jnp.maximum(
        jnp.dot(ea_ref[...].astype(bf), w1_ref[...].astype(bf),
                preferred_element_type=f32) + b1_ref[...], 0.0).astype(bf)
    w2 = w2_ref[...].astype(bf)
    s_m = s_ref[...].astype(bf)
    b2 = b2_ref[...].astype(bf)
    gbf = g_ref[...].astype(bf)
    wes = [jnp.dot(abd[:, 128 * p:128 * p + 128], w2,
                   preferred_element_type=f32).astype(bf) for p in range(8)]
    prods = [jnp.tile(gbf[:, H * p:H * p + H], (1, H)) * (wes[p] + b2)
             for p in range(8)]
    for p in range(8):
        msg = jnp.dot(prods[p], s_m, preferred_element_type=f32)
        msg_ref[:, H * p:H * p + H] = jnp.where(rmask, msg, 0.0)


def _msg(ea32, g128, w1bd, b1bd, w2p, b2p, s2):
    return pl.pallas_call(
        _msg_body,
        grid=(EP // 8 // MB,),
        in_specs=[
            pl.BlockSpec((MB, 32), lambda i: (i, 0)),
            pl.BlockSpec((MB, 128), lambda i: (i, 0)),
            pl.BlockSpec((32, 1024), lambda i: (0, 0)),
            pl.BlockSpec((1, 1024), lambda i: (0, 0)),
            pl.BlockSpec((128, H * H), lambda i: (0, 0)),
            pl.BlockSpec((1, H * H), lambda i: (0, 0)),
            pl.BlockSpec((H * H, H), lambda i: (0, 0)),
        ],
        out_specs=pl.BlockSpec((MB, 128), lambda i: (i, 0)),
        out_shape=jax.ShapeDtypeStruct((EP // 8, 128), f32),
    )(ea32, g128, w1bd, b1bd, w2p, b2p, s2)


def _node_body(s_ref, a0_ref, a1_ref, wroot_ref, bconv_ref,
               wir_ref, wiz_ref, win_ref, bi_ref,
               whr_ref, whz_ref, whn_ref, bh_ref, o_ref):
    sv = s_ref[...]
    agg = a0_ref[...] + a1_ref[...]
    m = jnp.maximum(
        jnp.dot(sv, wroot_ref[...], preferred_element_type=f32) + agg
        + bconv_ref[...], 0.0)
    bi = bi_ref[...]
    bh = bh_ref[...]
    gir = jnp.dot(m, wir_ref[...], preferred_element_type=f32) + bi[:, :H]
    giz = jnp.dot(m, wiz_ref[...], preferred_element_type=f32) + bi[:, H:2 * H]
    gin = jnp.dot(m, win_ref[...], preferred_element_type=f32) + bi[:, 2 * H:]
    ghr = jnp.dot(sv, whr_ref[...], preferred_element_type=f32) + bh[:, :H]
    ghz = jnp.dot(sv, whz_ref[...], preferred_element_type=f32) + bh[:, H:2 * H]
    ghn = jnp.dot(sv, whn_ref[...], preferred_element_type=f32) + bh[:, 2 * H:]
    r = jax.nn.sigmoid(gir + ghr)
    z = jax.nn.sigmoid(giz + ghz)
    n = jnp.tanh(gin + r * ghn)
    o_ref[...] = (1.0 - z) * n + z * sv


def _node(s, agg0, agg1, wroot_t, bconv, wir, wiz, win, bi, whr, whz, whn, bh):
    wspec = pl.BlockSpec((H, H), lambda i: (0, 0))
    bspec = pl.BlockSpec((1, 3 * H), lambda i: (0, 0))
    nspec = pl.BlockSpec((BLK, H), lambda i: (i, 0))
    return pl.pallas_call(
        _node_body,
        grid=(NBLK,),
        in_specs=[
            nspec, nspec, nspec,
            wspec, pl.BlockSpec((1, H), lambda i: (0, 0)),
            wspec, wspec, wspec, bspec,
            wspec, wspec, wspec, bspec,
        ],
        out_specs=nspec,
        out_shape=jax.ShapeDtypeStruct((NP, H), f32),
    )(s, agg0, agg1, wroot_t, bconv, wir, wiz, win, bi, whr, whz, whn, bh)


def _epi_body(s_ref, x_ref, bat_ref, w1_ref, b1_ref, w2_ref, b2_ref,
              fg1_ref, fg2_ref, seg1_ref, seg2_ref, cnt_ref):
    i = pl.program_id(0)
    hmid = jnp.maximum(
        jnp.dot(s_ref[...], w1_ref[...], preferred_element_type=f32)
        + b1_ref[...], 0.0)
    emb = jnp.dot(hmid, w2_ref[...], preferred_element_type=f32) + b2_ref[...]
    xv = x_ref[...]
    ss = (jnp.sum(emb * emb, axis=1, keepdims=True)
          + jnp.sum(xv * xv, axis=1, keepdims=True))
    inv = 1.0 / jnp.maximum(jnp.sqrt(ss), 1e-12)
    fg1 = emb * inv
    fg2 = xv * inv
    fg1_ref[...] = fg1
    fg2_ref[...] = fg2
    onehot = (bat_ref[...] == lax.broadcasted_iota(jnp.int32, (BLK, B), 1)
              ).astype(f32)
    dn = (((0,), (0,)), ((), ()))
    p1 = lax.dot_general(onehot, fg1, dn, preferred_element_type=f32)
    p2 = lax.dot_general(onehot, fg2, dn, preferred_element_type=f32)
    pc = jnp.broadcast_to(jnp.sum(onehot, axis=0)[:, None], (B, IN))

    @pl.when(i == 0)
    def _():
        seg1_ref[...] = jnp.zeros_like(seg1_ref)
        seg2_ref[...] = jnp.zeros_like(seg2_ref)
        cnt_ref[...] = jnp.zeros_like(cnt_ref)

    seg1_ref[...] += p1
    seg2_ref[...] += p2
    cnt_ref[...] += pc


def _epilogue(s, x, bat2, w1_t, b1, w2_t, b2):
    zspec = lambda shape: pl.BlockSpec(shape, lambda i: (0, 0))
    return pl.pallas_call(
        _epi_body,
        grid=(NBLK,),
        in_specs=[
            pl.BlockSpec((BLK, H), lambda i: (i, 0)),
            pl.BlockSpec((BLK, IN), lambda i: (i, 0)),
            pl.BlockSpec((BLK, 1), lambda i: (i, 0)),
            zspec((H, H)), zspec((1, H)), zspec((H, EMB)), zspec((1, EMB)),
        ],
        out_specs=[
            pl.BlockSpec((BLK, EMB), lambda i: (i, 0)),
            pl.BlockSpec((BLK, IN), lambda i: (i, 0)),
            zspec((B, EMB)), zspec((B, IN)), zspec((B, IN)),
        ],
        out_shape=[
            jax.ShapeDtypeStruct((NP, EMB), f32),
            jax.ShapeDtypeStruct((NP, IN), f32),
            jax.ShapeDtypeStruct((B, EMB), f32),
            jax.ShapeDtypeStruct((B, IN), f32),
            jax.ShapeDtypeStruct((B, IN), f32),
        ],
    )(s, x, bat2, w1_t, b1, w2_t, b2)


def _ratio_body(s1_ref, s2_ref, cnt_ref, wp1_ref, wp2_ref, bp_ref, o_ref):
    cnt = jnp.maximum(cnt_ref[...], 1.0)
    r1 = s1_ref[...] / cnt[:, :EMB]
    r2 = s2_ref[...] / cnt
    v = (jnp.dot(r1, wp1_ref[...], preferred_element_type=f32)
         + jnp.dot(r2, wp2_ref[...], preferred_element_type=f32)
         + bp_ref[...])
    o_ref[...] = jax.nn.sigmoid(v)


def _ratio(seg1, seg2, cnt, wp1_t, wp2_t, bp):
    return pl.pallas_call(
        _ratio_body,
        out_shape=jax.ShapeDtypeStruct((B, 1), f32),
    )(seg1, seg2, cnt, wp1_t, wp2_t, bp)


def kernel(x, edge_attr, W_in_fc, b_in_fc, W_e1, b_e1, W_e2, b_e2,
           W_root, b_conv, W_ih, W_hh, b_ih, b_hh,
           W_o1, b_o1, W_o2, b_o2, W_p, b_p,
           edge_index, batch):
    src = jnp.zeros((EP,), jnp.int32).at[:E].set(edge_index[0])
    dst = jnp.zeros((EP,), jnp.int32).at[:E].set(edge_index[1])
    src3 = src.reshape(NW, NCH, CH)
    dst3 = dst.reshape(NW, NCH, CH)
    ea32 = jnp.zeros((EP // 8, 32), f32).at[:E // 8].set(
        edge_attr.T.reshape(4, E // 8, 8).transpose(1, 2, 0).reshape(E // 8, 32))
    zero_n = jnp.zeros((NP, H), f32)
    x_p = jnp.zeros((NP, IN), f32).at[:N].set(x)
    bat2 = jnp.full((NP, 1), B, jnp.int32).at[:N, 0].set(batch)

    s2 = jnp.repeat(jnp.eye(H, dtype=f32), H, axis=0)

    w_in_t = W_in_fc.T
    w1bd = jax.scipy.linalg.block_diag(*([W_e1.T] * 8))
    b1bd = jnp.tile(b_e1.reshape(1, 128), (1, 8))
    w2p = W_e2.T.reshape(128, H, H).transpose(0, 2, 1).reshape(128, H * H)
    b2p = b_e2.reshape(H, H).T.reshape(1, H * H)
    wroot_t = W_root.T
    wir, wiz, win = (W_ih[:H].T, W_ih[H:2 * H].T, W_ih[2 * H:].T)
    whr, whz, whn = (W_hh[:H].T, W_hh[H:2 * H].T, W_hh[2 * H:].T)
    wo1_t = W_o1.T
    wo2_t = W_o2.T
    wp1_t = W_p[:, :EMB].T
    wp2_t = W_p[:, EMB:].T

    b_in = b_in_fc.reshape(1, H)
    bconv = b_conv.reshape(1, H)
    bi = b_ih.reshape(1, 3 * H)
    bh = b_hh.reshape(1, 3 * H)
    bo1 = b_o1.reshape(1, H)
    bo2 = b_o2.reshape(1, EMB)
    bp = b_p.reshape(1, 1)

    s = _in_fc(x_p, w_in_t, b_in)
    for _ in range(ITERS):
        g = _sc_gather(s, src3)
        msg128 = _msg(ea32, g.reshape(EP // 8, 128), w1bd, b1bd, w2p, b2p,
                      s2)
        agg2 = _sc_scatter(msg128.reshape(EP, H), dst3, zero_n)
        s = _node(s, agg2[0], agg2[1], wroot_t, bconv, wir, wiz, win, bi,
                  whr, whz, whn, bh)
    fg1, fg2, seg1, seg2, cnt = _epilogue(s, x_p, bat2, wo1_t, bo1, wo2_t, bo2)
    fg_embed = jnp.concatenate([fg1[:N], fg2[:N]], axis=1)
    cg_fg_ratio = _ratio(seg1, seg2, cnt, wp1_t, wp2_t, bp)
    return (fg_embed, cg_fg_ratio)
